# baseline (speedup 1.0000x reference)
import functools

import jax
import jax.numpy as jnp
from jax import lax
from jax.experimental import pallas as pl
from jax.experimental.pallas import tpu as pltpu
from jax.experimental.pallas import tpu_sc as plsc

NC = 2
NS = 16
L = 16
CH = 128
H = 64

_F32 = jnp.float32
_I32 = jnp.int32
_FILL = 1 << 28


def _ceil_to(x, m):
    return (x + m - 1) // m * m


def _mesh():
    return plsc.VectorSubcoreMesh(
        core_axis_name="c", subcore_axis_name="s", num_cores=NC, num_subcores=NS
    )


def _sc_params():
    return pltpu.CompilerParams(use_tc_tiling_on_sc=False)


def _dot(a, b):
    return jax.lax.dot_general(
        a, b, (((1,), (0,)), ((), ())), preferred_element_type=jnp.float32
    )


@functools.lru_cache(maxsize=None)
def _make_segsum(n_src, n_dst, e_pad):
    nh = _ceil_to((n_dst + 1) // 2, CH)
    acc_rows = nh + CH
    ept = e_pad // NS
    nb = ept // CH
    nbh = nb // 2
    assert nb % 2 == 0
    rpt_z = acc_rows // NS
    rpt_w = nh // NS

    @functools.partial(
        pl.kernel,
        out_type=jax.ShapeDtypeStruct((2 * nh, H), _F32),
        mesh=_mesh(),
        compiler_params=_sc_params(),
        scratch_types=[
            pltpu.VMEM((2 * CH,), _I32),
            pltpu.VMEM((2 * CH,), _I32),
            pltpu.VMEM((1, CH), _I32),
            pltpu.VMEM((1, CH), _I32),
            pltpu.VMEM((CH, H), _F32),
            pltpu.VMEM((CH, H), _F32),
            pltpu.VMEM_SHARED((acc_rows, H), _F32),
            pltpu.SemaphoreType.DMA,
            pltpu.SemaphoreType.DMA,
        ],
    )
    def seg(
        hsrc, sd_hbm, zrow, out,
        sd0, sd1, lidx0, lidx1, rows0, rows1, acc, semg, sems,
    ):
        cid = lax.axis_index("c")
        tid = lax.axis_index("s")

        pltpu.sync_copy(zrow, rows0)
        z0 = tid * rpt_z
        zf, zr = divmod(rpt_z, CH)
        for k in range(zf):
            pltpu.sync_copy(rows0, acc.at[pl.ds(z0 + k * CH, CH)])
        if zr:
            pltpu.sync_copy(rows0.at[pl.ds(0, zr)], acc.at[pl.ds(z0 + zf * CH, zr)])
        plsc.subcore_barrier()

        base = cid * nh
        dummy = jnp.int32(nh)
        t0 = tid * nb

        def load_idx(blk, sd, lidx):
            pltpu.sync_copy(sd_hbm.at[pl.ds((t0 + blk) * 2 * CH, 2 * CH)], sd)
            for i in range(CH // L):
                d = sd[pl.ds(CH + i * L, L)]
                ld = d - base
                ok = (ld >= 0) & (ld < nh)
                spill = dummy + (ld & (CH - 1))
                lidx[0, pl.ds(i * L, L)] = jnp.where(ok, ld, spill)

        def issue_g(sd, rows):
            pltpu.async_copy(hsrc.at[sd.at[pl.ds(0, CH)]], rows, semg)

        def drain_g(sd, rows):
            pltpu.make_async_copy(hsrc.at[sd.at[pl.ds(0, CH)]], rows, semg).wait()

        def issue_s(lidx, rows):
            pltpu.async_copy(rows, acc.at[lidx.at[0]], sems, add=True)

        def drain_s(lidx, rows):
            pltpu.make_async_copy(rows, acc.at[lidx.at[0]], sems).wait()

        load_idx(0, sd0, lidx0)
        issue_g(sd0, rows0)

        def body(j, carry):
            b = 2 * j
            drain_g(sd0, rows0)
            issue_s(lidx0, rows0)

            @pl.when(j > 0)
            def _():
                drain_s(lidx1, rows1)

            load_idx(b + 1, sd1, lidx1)
            issue_g(sd1, rows1)
            drain_s(lidx0, rows0)

            @pl.when(j < nbh - 1)
            def _():
                load_idx(b + 2, sd0, lidx0)
                issue_g(sd0, rows0)

            drain_g(sd1, rows1)
            issue_s(lidx1, rows1)
            return carry

        lax.fori_loop(0, nbh, body, 0)
        drain_s(lidx1, rows1)
        plsc.subcore_barrier()

        w0 = tid * rpt_w
        wf, wr = divmod(rpt_w, CH)
        for k in range(wf):
            pltpu.sync_copy(acc.at[pl.ds(w0 + k * CH, CH)], rows0)
            pltpu.sync_copy(rows0, out.at[pl.ds(base + w0 + k * CH, CH)])
        if wr:
            pltpu.sync_copy(acc.at[pl.ds(w0 + wf * CH, wr)], rows0.at[pl.ds(0, wr)])
            pltpu.sync_copy(
                rows0.at[pl.ds(0, wr)], out.at[pl.ds(base + w0 + wf * CH, wr)]
            )

    return seg


def _segsum(hsrc, sd_packed, n_dst, zrow):
    return _make_segsum(hsrc.shape[0], n_dst, sd_packed.shape[0] // 2)(
        hsrc, sd_packed, zrow
    )


@functools.lru_cache(maxsize=None)
def _make_counts(dir_shapes):
    rowoffs = []
    r = 0
    for n_dst, _ in dir_shapes:
        rowoffs.append(r)
        r += -(-n_dst // L)
    rr = _ceil_to(r, CH)
    acc_rows = rr + CH
    rpt_z = acc_rows // NS
    rpt_w = rr // NS

    @functools.partial(
        pl.kernel,
        out_type=jax.ShapeDtypeStruct((2 * rr, L), _F32),
        mesh=_mesh(),
        compiler_params=_sc_params(),
        scratch_types=[
            pltpu.VMEM((CH,), _I32),
            pltpu.VMEM((CH,), _I32),
            pltpu.VMEM((1, CH), _I32),
            pltpu.VMEM((1, CH), _I32),
            pltpu.VMEM((CH, L), _F32),
            pltpu.VMEM((CH, L), _F32),
            pltpu.VMEM_SHARED((acc_rows, L), _F32),
            pltpu.SemaphoreType.DMA,
        ],
    )
    def cnt(
        d1, d2, d3, d4, zrow, out,
        dd0, dd1, lidx0, lidx1, rows0, rows1, acc, sems,
    ):
        cid = lax.axis_index("c")
        tid = lax.axis_index("s")

        pltpu.sync_copy(zrow, rows0)
        z0 = tid * rpt_z
        zf, zr = divmod(rpt_z, CH)
        for k in range(zf):
            pltpu.sync_copy(rows0, acc.at[pl.ds(z0 + k * CH, CH)])
        if zr:
            pltpu.sync_copy(rows0.at[pl.ds(0, zr)], acc.at[pl.ds(z0 + zf * CH, zr)])
        plsc.subcore_barrier()

        bound = jnp.int32(r)
        spill0 = jnp.int32(rr)
        iota = lax.iota(_I32, L)

        def issue_s(lidx, rows):
            pltpu.async_copy(rows, acc.at[lidx.at[0]], sems, add=True)

        def drain_s(lidx, rows):
            pltpu.make_async_copy(rows, acc.at[lidx.at[0]], sems).wait()

        for d_ref, (n_dst, e_pad), rowoff in zip((d1, d2, d3, d4), dir_shapes, rowoffs):
            ept = e_pad // (NS * NC)
            nb = ept // CH
            nbh = nb // 2

            def load_idx(blk, dd, lidx, rows, d_ref=d_ref, ept=ept, rowoff=rowoff, e_pad=e_pad):
                o = cid * (e_pad // NC) + tid * ept + blk * CH
                pltpu.sync_copy(d_ref.at[pl.ds(o, CH)], dd)
                for i in range(CH // L):
                    d = dd[pl.ds(i * L, L)]
                    m = d & 15
                    lr = (d >> 4) + rowoff
                    ok = lr < bound
                    spill = spill0 + (lr & (CH - 1))
                    lidx[0, pl.ds(i * L, L)] = jnp.where(ok, lr, spill)
                    for q in range(L):
                        mb = jnp.broadcast_to(m[q], (L,))
                        rows[i * L + q, pl.ds(0, L)] = jnp.where(
                            iota == mb, 1.0, 0.0
                        )

            load_idx(0, dd0, lidx0, rows0)

            def body(j, carry, load_idx=load_idx, nbh=nbh):
                b = 2 * j
                issue_s(lidx0, rows0)

                @pl.when(j > 0)
                def _():
                    drain_s(lidx1, rows1)

                load_idx(b + 1, dd1, lidx1, rows1)
                issue_s(lidx1, rows1)
                drain_s(lidx0, rows0)

                @pl.when(j < nbh - 1)
                def _():
                    load_idx(b + 2, dd0, lidx0, rows0)
                return carry

            lax.fori_loop(0, nbh, body, 0)
            drain_s(lidx1, rows1)
        plsc.subcore_barrier()

        w0 = tid * rpt_w
        wf, wr = divmod(rpt_w, CH)
        obase = cid * rr
        for k in range(wf):
            pltpu.sync_copy(acc.at[pl.ds(w0 + k * CH, CH)], rows0)
            pltpu.sync_copy(rows0, out.at[pl.ds(obase + w0 + k * CH, CH)])
        if wr:
            pltpu.sync_copy(acc.at[pl.ds(w0 + wf * CH, wr)], rows0.at[pl.ds(0, wr)])
            pltpu.sync_copy(
                rows0.at[pl.ds(0, wr)], out.at[pl.ds(obase + w0 + wf * CH, wr)]
            )

    return cnt, (tuple(rowoffs), rr)


@functools.lru_cache(maxsize=None)
def _make_gather4(n_c_rows, n_p_rows, e_pad):
    ept = e_pad // (NS * NC)
    nb = ept // CH
    nbh = nb // 2
    assert nb % 2 == 0

    @functools.partial(
        pl.kernel,
        out_type=[jax.ShapeDtypeStruct((e_pad, H), _F32) for _ in range(4)],
        mesh=_mesh(),
        compiler_params=_sc_params(),
        scratch_types=[
            pltpu.VMEM((CH,), _I32),
            pltpu.VMEM((CH,), _I32),
            pltpu.VMEM((CH, H), _F32),
            pltpu.VMEM((CH, H), _F32),
            pltpu.SemaphoreType.DMA,
            pltpu.SemaphoreType.DMA,
        ],
    )
    def gat(
        hc, hp, i1, i2, i3, i4, o1, o2, o3, o4,
        sidx0, sidx1, rows0, rows1, semg, semw,
    ):
        cid = lax.axis_index("c")
        tid = lax.axis_index("s")
        wid = tid * NC + cid
        t0 = wid * ept

        def run(t_r, i_r, o_r):
            def load_idx(blk, sidx):
                pltpu.sync_copy(i_r.at[pl.ds(t0 + blk * CH, CH)], sidx)

            def issue_g(sidx, rows):
                pltpu.async_copy(t_r.at[sidx], rows, semg)

            def drain_g(sidx, rows):
                pltpu.make_async_copy(t_r.at[sidx], rows, semg).wait()

            def issue_w(blk, rows):
                pltpu.async_copy(rows, o_r.at[pl.ds(t0 + blk * CH, CH)], semw)

            def drain_w(blk, rows):
                pltpu.make_async_copy(
                    rows, o_r.at[pl.ds(t0 + blk * CH, CH)], semw
                ).wait()

            load_idx(0, sidx0)
            issue_g(sidx0, rows0)

            def body(j, carry):
                b = 2 * j
                drain_g(sidx0, rows0)
                issue_w(b, rows0)

                @pl.when(j > 0)
                def _():
                    drain_w(b - 1, rows1)

                load_idx(b + 1, sidx1)
                issue_g(sidx1, rows1)
                drain_w(b, rows0)

                @pl.when(j < nbh - 1)
                def _():
                    load_idx(b + 2, sidx0)
                    issue_g(sidx0, rows0)

                drain_g(sidx1, rows1)
                issue_w(b + 1, rows1)
                return carry

            lax.fori_loop(0, nbh, body, 0)
            drain_w(nb - 1, rows1)

        run(hc, i1, o1)
        run(hp, i2, o2)
        run(hc, i3, o3)
        run(hp, i4, o4)

    return gat


def _embed(x, w1, b1, w2, b2, blk):
    n, d = x.shape

    def body(x_r, w1_r, b1_r, w2_r, b2_r, o_r):
        h = _dot(x_r[...], w1_r[...]) + b1_r[...]
        h = _dot(h, w2_r[...]) + b2_r[...]
        o_r[...] = jnp.where(h >= 0, h, 0.01 * h)

    return pl.pallas_call(
        body,
        grid=(n // blk,),
        in_specs=[
            pl.BlockSpec((blk, d), lambda i: (i, 0)),
            pl.BlockSpec((d, H), lambda i: (0, 0)),
            pl.BlockSpec((1, H), lambda i: (0, 0)),
            pl.BlockSpec((H, H), lambda i: (0, 0)),
            pl.BlockSpec((1, H), lambda i: (0, 0)),
        ],
        out_specs=pl.BlockSpec((blk, H), lambda i: (i, 0)),
        out_shape=jax.ShapeDtypeStruct((n, H), _F32),
    )(x, w1, b1.reshape(1, H), w2, b2.reshape(1, H))


def _combine(h_dst, parts, blk):
    n = h_dst.shape[0]
    nparts = len(parts)

    def body(*refs):
        hd = refs[0][...]
        o_r = refs[-1]
        acc = None
        for k in range(nparts):
            s_r, c1_r, c2_r, ws_r, wn_r = refs[1 + 5 * k : 6 + 5 * k]
            cc = jnp.maximum(c1_r[...] + c2_r[...], 1.0)
            agg = s_r[...] / cc
            z = jnp.maximum(_dot(hd, ws_r[...]) + _dot(agg, wn_r[...]), 0.0)
            nn = jnp.sqrt(jnp.sum(z * z, axis=1, keepdims=True))
            nn = jnp.where(nn == 0.0, 1.0, nn)
            zn = z / nn
            acc = zn if acc is None else acc + zn
        o_r[...] = acc

    in_specs = [pl.BlockSpec((blk, H), lambda i: (i, 0))]
    args = [h_dst]
    for s, (c1, c2), ws, wn in parts:
        in_specs += [
            pl.BlockSpec((blk, H), lambda i: (i, 0)),
            pl.BlockSpec((blk, 1), lambda i: (i, 0)),
            pl.BlockSpec((blk, 1), lambda i: (i, 0)),
            pl.BlockSpec((H, H), lambda i: (0, 0)),
            pl.BlockSpec((H, H), lambda i: (0, 0)),
        ]
        args += [s, c1, c2, ws, wn]
    return pl.pallas_call(
        body,
        grid=(n // blk,),
        in_specs=in_specs,
        out_specs=pl.BlockSpec((blk, H), lambda i: (i, 0)),
        out_shape=jax.ShapeDtypeStruct((n, H), _F32),
    )(*args)


def _selu(x):
    return 1.0507009873554805 * jnp.where(
        x > 0, x, 1.6732632423543772 * (jnp.exp(x) - 1.0)
    )


def _pred_mlp(xs, xd, w1, b1, w2, b2, w3, b3, blk):
    n = xs.shape[0]

    def body(xs_r, xd_r, w1_r, b1_r, w2_r, b2_r, w3_r, b3_r, o_r):
        w1v = w1_r[...]
        x = _dot(xs_r[...], w1v[0:H]) + _dot(xd_r[...], w1v[H : 2 * H]) + b1_r[...]
        x = _selu(x)
        x = _selu(_dot(x, w2_r[...]) + b2_r[...])
        o_r[...] = _dot(x, w3_r[...]) + b3_r[...]

    return pl.pallas_call(
        body,
        grid=(n // blk,),
        in_specs=[
            pl.BlockSpec((blk, H), lambda i: (i, 0)),
            pl.BlockSpec((blk, H), lambda i: (i, 0)),
            pl.BlockSpec((2 * H, 16), lambda i: (0, 0)),
            pl.BlockSpec((1, 16), lambda i: (0, 0)),
            pl.BlockSpec((16, 8), lambda i: (0, 0)),
            pl.BlockSpec((1, 8), lambda i: (0, 0)),
            pl.BlockSpec((8, 1), lambda i: (0, 0)),
            pl.BlockSpec((1, 1), lambda i: (0, 0)),
        ],
        out_specs=pl.BlockSpec((blk, 1), lambda i: (i, 0)),
        out_shape=jax.ShapeDtypeStruct((n, 1), _F32),
    )(
        xs,
        xd,
        w1,
        b1.reshape(1, 16),
        w2,
        b2.reshape(1, 8),
        w3,
        b3.reshape(1, 1),
    )


def _pad_idx(a, m, fill):
    e = a.shape[0]
    ep = _ceil_to(e, m)
    if ep != e:
        a = jnp.concatenate([a, jnp.full((ep - e,), fill, _I32)])
    return a


def _pack_sd(src, dst, m):
    s = _pad_idx(src, m, 0).reshape(-1, 1, CH)
    d = _pad_idx(dst, m, _FILL).reshape(-1, 1, CH)
    return jnp.concatenate([s, d], axis=1).reshape(-1)


def kernel(
    customer_feats,
    product_feats,
    group_feats,
    edge_cp_src,
    edge_cp_dst,
    edge_pg_src,
    edge_pg_dst,
    pos_src,
    pos_dst,
    neg_src,
    neg_dst,
    params,
):
    p = params
    n_c = customer_feats.shape[0]
    n_p = product_feats.shape[0]
    n_g = group_feats.shape[0]

    cp_s = edge_cp_src.astype(_I32)
    cp_d = edge_cp_dst.astype(_I32)
    pg_s = edge_pg_src.astype(_I32)
    pg_d = edge_pg_dst.astype(_I32)

    m = NS * NC * CH * 2
    sd_cp = _pack_sd(cp_s, cp_d, m)
    sd_pc = _pack_sd(cp_d, cp_s, m)
    sd_gp = _pack_sd(pg_d, pg_s, m)
    sd_pg = _pack_sd(pg_s, pg_d, m)
    cp_dF = _pad_idx(cp_d, m, _FILL)
    cp_sF = _pad_idx(cp_s, m, _FILL)
    pg_sF = _pad_idx(pg_s, m, _FILL)
    pg_dF = _pad_idx(pg_d, m, _FILL)

    zrow_h = jnp.zeros((CH, H), _F32)
    zrow_l = jnp.zeros((CH, L), _F32)

    h_c = _embed(customer_feats, p["Wc1"], p["bc1"], p["Wc2"], p["bc2"], 2000)
    h_p = _embed(product_feats, p["Wp1e"], p["bp1e"], p["Wp2e"], p["bp2e"], 2000)
    h_g = _embed(group_feats, p["Wg1"], p["bg1"], p["Wg2"], p["bg2"], 1000)

    dir_shapes = (
        (n_p, cp_dF.shape[0]),
        (n_c, cp_sF.shape[0]),
        (n_p, pg_sF.shape[0]),
        (n_g, pg_dF.shape[0]),
    )
    cnt_kernel, (rowoffs, rr) = _make_counts(dir_shapes)
    cnt_packed = cnt_kernel(cp_dF, cp_sF, pg_sF, pg_dF, zrow_l)
    cnt_flat = cnt_packed.reshape(-1)
    counts = []
    for (n_dst, _), ro in zip(dir_shapes, rowoffs):
        mrows = -(-n_dst // L)
        c1 = cnt_flat[ro * L : ro * L + mrows * L][:n_dst].reshape(n_dst, 1)
        o2 = (rr + ro) * L
        c2 = cnt_flat[o2 : o2 + mrows * L][:n_dst].reshape(n_dst, 1)
        counts.append((c1, c2))
    cnt_cp, cnt_pc, cnt_gp, cnt_pg = counts

    for l in range(2):
        s_cp = _segsum(h_c, sd_cp, n_p, zrow_h)
        s_gp = _segsum(h_g, sd_gp, n_p, zrow_h)
        s_pc = _segsum(h_p, sd_pc, n_c, zrow_h)
        s_pg = _segsum(h_p, sd_pg, n_g, zrow_h)
        z_p = _combine(
            h_p,
            [
                (s_cp, cnt_cp, p["Ws%d_cp" % l], p["Wn%d_cp" % l]),
                (s_gp, cnt_gp, p["Ws%d_gp" % l], p["Wn%d_gp" % l]),
            ],
            2000,
        )
        z_c = _combine(
            h_c, [(s_pc, cnt_pc, p["Ws%d_pc" % l], p["Wn%d_pc" % l])], 2000
        )
        z_g = _combine(
            h_g, [(s_pg, cnt_pg, p["Ws%d_pg" % l], p["Wn%d_pg" % l])], 1000
        )
        h_c, h_p, h_g = z_c, z_p, z_g

    e_pos = pos_src.shape[0]
    e_neg = neg_src.shape[0]
    mg = NS * NC * CH * 2
    ps = _pad_idx(pos_src.astype(_I32), mg, 0)
    pd = _pad_idx(pos_dst.astype(_I32), mg, 0)
    ns_ = _pad_idx(neg_src.astype(_I32), mg, 0)
    nd = _pad_idx(neg_dst.astype(_I32), mg, 0)
    e_pad = ps.shape[0]
    g1, g2, g3, g4 = _make_gather4(n_c, n_p, e_pad)(h_c, h_p, ps, pd, ns_, nd)

    pos = _pred_mlp(
        g1, g2, p["Wq1"], p["bq1"], p["Wq2"], p["bq2"], p["Wq3"], p["bq3"], 2048
    )[:e_pos]
    neg = _pred_mlp(
        g3, g4, p["Wq1"], p["bq1"], p["Wq2"], p["bq2"], p["Wq3"], p["bq3"], 2048
    )[:e_neg]

    return h_c, h_p, h_g, pos, neg


# speedup vs baseline: 1.0621x; 1.0621x over previous
import functools

import jax
import jax.numpy as jnp
from jax import lax
from jax.experimental import pallas as pl
from jax.experimental.pallas import tpu as pltpu
from jax.experimental.pallas import tpu_sc as plsc

NC = 2
NS = 16
L = 16
CH = 128
H = 64

_F32 = jnp.float32
_I32 = jnp.int32
_FILL = 1 << 28


def _ceil_to(x, m):
    return (x + m - 1) // m * m


def _mesh():
    return plsc.VectorSubcoreMesh(
        core_axis_name="c", subcore_axis_name="s", num_cores=NC, num_subcores=NS
    )


def _sc_params():
    return pltpu.CompilerParams(use_tc_tiling_on_sc=False)


def _dot(a, b):
    return jax.lax.dot_general(
        a, b, (((1,), (0,)), ((), ())), preferred_element_type=jnp.float32
    )


@functools.lru_cache(maxsize=None)
def _make_segsum(n_src, n_dst, e_pad):
    nh = _ceil_to((n_dst + 1) // 2, CH)
    acc_rows = nh + CH
    ept = e_pad // NS
    nb = ept // CH
    nbh = nb // 2
    assert nb % 2 == 0
    rpt_z = acc_rows // NS
    rpt_w = nh // NS

    @functools.partial(
        pl.kernel,
        out_type=jax.ShapeDtypeStruct((2 * nh, H), _F32),
        mesh=_mesh(),
        compiler_params=_sc_params(),
        scratch_types=[
            pltpu.VMEM((2 * CH,), _I32),
            pltpu.VMEM((2 * CH,), _I32),
            pltpu.VMEM((1, CH), _I32),
            pltpu.VMEM((1, CH), _I32),
            pltpu.VMEM((CH, H), _F32),
            pltpu.VMEM((CH, H), _F32),
            pltpu.VMEM_SHARED((acc_rows, H), _F32),
            pltpu.SemaphoreType.DMA,
            pltpu.SemaphoreType.DMA,
            pltpu.SemaphoreType.DMA,
            pltpu.SemaphoreType.DMA,
        ],
    )
    def seg(
        hsrc, sd_hbm, zrow, out,
        sd0, sd1, lidx0, lidx1, rows0, rows1, acc, semg0, semg1, sems0, sems1,
    ):
        cid = lax.axis_index("c")
        tid = lax.axis_index("s")

        pltpu.sync_copy(zrow, rows0)
        z0 = tid * rpt_z
        zf, zr = divmod(rpt_z, CH)
        for k in range(zf):
            pltpu.sync_copy(rows0, acc.at[pl.ds(z0 + k * CH, CH)])
        if zr:
            pltpu.sync_copy(rows0.at[pl.ds(0, zr)], acc.at[pl.ds(z0 + zf * CH, zr)])
        plsc.subcore_barrier()

        base = cid * nh
        dummy = jnp.int32(nh)
        t0 = tid * nb

        def load_idx(blk, sd, lidx):
            pltpu.sync_copy(sd_hbm.at[pl.ds((t0 + blk) * 2 * CH, 2 * CH)], sd)
            for i in range(CH // L):
                d = sd[pl.ds(CH + i * L, L)]
                ld = d - base
                ok = (ld >= 0) & (ld < nh)
                spill = dummy + (ld & (CH - 1))
                lidx[0, pl.ds(i * L, L)] = jnp.where(ok, ld, spill)

        def issue_g(sd, rows, semg):
            pltpu.async_copy(hsrc.at[sd.at[pl.ds(0, CH)]], rows, semg)

        def drain_g(sd, rows, semg):
            pltpu.make_async_copy(hsrc.at[sd.at[pl.ds(0, CH)]], rows, semg).wait()

        def issue_s(lidx, rows, sems):
            pltpu.async_copy(rows, acc.at[lidx.at[0]], sems, add=True)

        def drain_s(lidx, rows, sems):
            pltpu.make_async_copy(rows, acc.at[lidx.at[0]], sems).wait()

        load_idx(0, sd0, lidx0)
        issue_g(sd0, rows0, semg0)

        def body(j, carry):
            b = 2 * j
            drain_g(sd0, rows0, semg0)
            issue_s(lidx0, rows0, sems0)

            @pl.when(j > 0)
            def _():
                drain_s(lidx1, rows1, sems1)

            load_idx(b + 1, sd1, lidx1)
            issue_g(sd1, rows1, semg1)
            drain_s(lidx0, rows0, sems0)

            @pl.when(j < nbh - 1)
            def _():
                load_idx(b + 2, sd0, lidx0)
                issue_g(sd0, rows0, semg0)

            drain_g(sd1, rows1, semg1)
            issue_s(lidx1, rows1, sems1)
            return carry

        lax.fori_loop(0, nbh, body, 0)
        drain_s(lidx1, rows1, sems1)
        plsc.subcore_barrier()

        w0 = tid * rpt_w
        wf, wr = divmod(rpt_w, CH)
        for k in range(wf):
            pltpu.sync_copy(acc.at[pl.ds(w0 + k * CH, CH)], rows0)
            pltpu.sync_copy(rows0, out.at[pl.ds(base + w0 + k * CH, CH)])
        if wr:
            pltpu.sync_copy(acc.at[pl.ds(w0 + wf * CH, wr)], rows0.at[pl.ds(0, wr)])
            pltpu.sync_copy(
                rows0.at[pl.ds(0, wr)], out.at[pl.ds(base + w0 + wf * CH, wr)]
            )

    return seg


def _segsum(hsrc, sd_packed, n_dst, zrow):
    return _make_segsum(hsrc.shape[0], n_dst, sd_packed.shape[0] // 2)(
        hsrc, sd_packed, zrow
    )


@functools.lru_cache(maxsize=None)
def _make_counts(dir_shapes):
    rowoffs = []
    r = 0
    for n_dst, _ in dir_shapes:
        rowoffs.append(r)
        r += -(-n_dst // L)
    rh = _ceil_to((r + 1) // 2, CH)
    acc_rows = rh + CH
    rpt_z = acc_rows // NS
    rpt_w = rh // NS

    @functools.partial(
        pl.kernel,
        out_type=jax.ShapeDtypeStruct((2 * rh, L), _F32),
        mesh=_mesh(),
        compiler_params=_sc_params(),
        scratch_types=[
            pltpu.VMEM((CH,), _I32),
            pltpu.VMEM((CH,), _I32),
            pltpu.VMEM((1, CH), _I32),
            pltpu.VMEM((1, CH), _I32),
            pltpu.VMEM((CH, L), _F32),
            pltpu.VMEM((CH, L), _F32),
            pltpu.VMEM_SHARED((acc_rows, L), _F32),
            pltpu.SemaphoreType.DMA,
            pltpu.SemaphoreType.DMA,
        ],
    )
    def cnt(
        d1, d2, d3, d4, zrow, out,
        dd0, dd1, lidx0, lidx1, rows0, rows1, acc, sems0, sems1,
    ):
        cid = lax.axis_index("c")
        tid = lax.axis_index("s")

        pltpu.sync_copy(zrow, rows0)
        z0 = tid * rpt_z
        zf, zr = divmod(rpt_z, CH)
        for k in range(zf):
            pltpu.sync_copy(rows0, acc.at[pl.ds(z0 + k * CH, CH)])
        if zr:
            pltpu.sync_copy(rows0.at[pl.ds(0, zr)], acc.at[pl.ds(z0 + zf * CH, zr)])
        plsc.subcore_barrier()

        base = cid * rh
        dummy = jnp.int32(rh)
        iota = lax.iota(_I32, L)

        def issue_s(lidx, rows, sems):
            pltpu.async_copy(rows, acc.at[lidx.at[0]], sems, add=True)

        def drain_s(lidx, rows, sems):
            pltpu.make_async_copy(rows, acc.at[lidx.at[0]], sems).wait()

        for d_ref, (n_dst, e_pad), rowoff in zip((d1, d2, d3, d4), dir_shapes, rowoffs):
            ept = e_pad // NS
            nb = ept // CH
            nbh = nb // 2
            off = rowoff - base

            def load_idx(blk, dd, lidx, rows, d_ref=d_ref, ept=ept, off=off):
                pltpu.sync_copy(d_ref.at[pl.ds(tid * ept + blk * CH, CH)], dd)
                for i in range(CH // L):
                    d = dd[pl.ds(i * L, L)]
                    m = d & 15
                    lr = (d >> 4) + off
                    ok = (lr >= 0) & (lr < rh)
                    spill = dummy + (lr & (CH - 1))
                    lidx[0, pl.ds(i * L, L)] = jnp.where(ok, lr, spill)
                    for r in range(L):
                        mb = jnp.broadcast_to(m[r], (L,))
                        rows[i * L + r, pl.ds(0, L)] = jnp.where(
                            iota == mb, 1.0, 0.0
                        )

            load_idx(0, dd0, lidx0, rows0)

            def body(j, carry, load_idx=load_idx, nbh=nbh):
                b = 2 * j
                issue_s(lidx0, rows0, sems0)

                @pl.when(j > 0)
                def _():
                    drain_s(lidx1, rows1, sems1)

                load_idx(b + 1, dd1, lidx1, rows1)
                issue_s(lidx1, rows1, sems1)
                drain_s(lidx0, rows0, sems0)

                @pl.when(j < nbh - 1)
                def _():
                    load_idx(b + 2, dd0, lidx0, rows0)
                return carry

            lax.fori_loop(0, nbh, body, 0)
            drain_s(lidx1, rows1, sems1)
        plsc.subcore_barrier()

        w0 = tid * rpt_w
        wf, wr = divmod(rpt_w, CH)
        for k in range(wf):
            pltpu.sync_copy(acc.at[pl.ds(w0 + k * CH, CH)], rows0)
            pltpu.sync_copy(rows0, out.at[pl.ds(base + w0 + k * CH, CH)])
        if wr:
            pltpu.sync_copy(acc.at[pl.ds(w0 + wf * CH, wr)], rows0.at[pl.ds(0, wr)])
            pltpu.sync_copy(
                rows0.at[pl.ds(0, wr)], out.at[pl.ds(base + w0 + wf * CH, wr)]
            )

    return cnt, tuple(rowoffs)


@functools.lru_cache(maxsize=None)
def _make_gather4(n_c_rows, n_p_rows, e_pad):
    ept = e_pad // (NS * NC)
    nb = ept // CH
    nbh = nb // 2
    assert nb % 2 == 0

    @functools.partial(
        pl.kernel,
        out_type=[jax.ShapeDtypeStruct((e_pad, H), _F32) for _ in range(4)],
        mesh=_mesh(),
        compiler_params=_sc_params(),
        scratch_types=[
            pltpu.VMEM((CH,), _I32),
            pltpu.VMEM((CH,), _I32),
            pltpu.VMEM((CH, H), _F32),
            pltpu.VMEM((CH, H), _F32),
            pltpu.SemaphoreType.DMA,
            pltpu.SemaphoreType.DMA,
            pltpu.SemaphoreType.DMA,
            pltpu.SemaphoreType.DMA,
        ],
    )
    def gat(
        hc, hp, i1, i2, i3, i4, o1, o2, o3, o4,
        sidx0, sidx1, rows0, rows1, semg0, semg1, semw0, semw1,
    ):
        cid = lax.axis_index("c")
        tid = lax.axis_index("s")
        wid = tid * NC + cid
        t0 = wid * ept

        def run(t_r, i_r, o_r):
            def load_idx(blk, sidx):
                pltpu.sync_copy(i_r.at[pl.ds(t0 + blk * CH, CH)], sidx)

            def issue_g(sidx, rows, semg):
                pltpu.async_copy(t_r.at[sidx], rows, semg)

            def drain_g(sidx, rows, semg):
                pltpu.make_async_copy(t_r.at[sidx], rows, semg).wait()

            def issue_w(blk, rows, semw):
                pltpu.async_copy(rows, o_r.at[pl.ds(t0 + blk * CH, CH)], semw)

            def drain_w(blk, rows, semw):
                pltpu.make_async_copy(
                    rows, o_r.at[pl.ds(t0 + blk * CH, CH)], semw
                ).wait()

            load_idx(0, sidx0)
            issue_g(sidx0, rows0, semg0)

            def body(j, carry):
                b = 2 * j
                drain_g(sidx0, rows0, semg0)
                issue_w(b, rows0, semw0)

                @pl.when(j > 0)
                def _():
                    drain_w(b - 1, rows1, semw1)

                load_idx(b + 1, sidx1)
                issue_g(sidx1, rows1, semg1)
                drain_w(b, rows0, semw0)

                @pl.when(j < nbh - 1)
                def _():
                    load_idx(b + 2, sidx0)
                    issue_g(sidx0, rows0, semg0)

                drain_g(sidx1, rows1, semg1)
                issue_w(b + 1, rows1, semw1)
                return carry

            lax.fori_loop(0, nbh, body, 0)
            drain_w(nb - 1, rows1, semw1)

        run(hc, i1, o1)
        run(hp, i2, o2)
        run(hc, i3, o3)
        run(hp, i4, o4)

    return gat


def _embed(x, w1, b1, w2, b2, blk):
    n, d = x.shape

    def body(x_r, w1_r, b1_r, w2_r, b2_r, o_r):
        h = _dot(x_r[...], w1_r[...]) + b1_r[...]
        h = _dot(h, w2_r[...]) + b2_r[...]
        o_r[...] = jnp.where(h >= 0, h, 0.01 * h)

    return pl.pallas_call(
        body,
        grid=(n // blk,),
        in_specs=[
            pl.BlockSpec((blk, d), lambda i: (i, 0)),
            pl.BlockSpec((d, H), lambda i: (0, 0)),
            pl.BlockSpec((1, H), lambda i: (0, 0)),
            pl.BlockSpec((H, H), lambda i: (0, 0)),
            pl.BlockSpec((1, H), lambda i: (0, 0)),
        ],
        out_specs=pl.BlockSpec((blk, H), lambda i: (i, 0)),
        out_shape=jax.ShapeDtypeStruct((n, H), _F32),
    )(x, w1, b1.reshape(1, H), w2, b2.reshape(1, H))


def _combine(h_dst, parts, blk):
    n = h_dst.shape[0]
    nparts = len(parts)

    def body(*refs):
        hd = refs[0][...]
        o_r = refs[-1]
        acc = None
        for k in range(nparts):
            s_r, c_r, ws_r, wn_r = refs[1 + 4 * k : 5 + 4 * k]
            cc = jnp.maximum(c_r[...], 1.0)
            agg = s_r[...] / cc
            z = jnp.maximum(_dot(hd, ws_r[...]) + _dot(agg, wn_r[...]), 0.0)
            nn = jnp.sqrt(jnp.sum(z * z, axis=1, keepdims=True))
            nn = jnp.where(nn == 0.0, 1.0, nn)
            zn = z / nn
            acc = zn if acc is None else acc + zn
        o_r[...] = acc

    in_specs = [pl.BlockSpec((blk, H), lambda i: (i, 0))]
    args = [h_dst]
    for s, c, ws, wn in parts:
        in_specs += [
            pl.BlockSpec((blk, H), lambda i: (i, 0)),
            pl.BlockSpec((blk, 1), lambda i: (i, 0)),
            pl.BlockSpec((H, H), lambda i: (0, 0)),
            pl.BlockSpec((H, H), lambda i: (0, 0)),
        ]
        args += [s, c, ws, wn]
    return pl.pallas_call(
        body,
        grid=(n // blk,),
        in_specs=in_specs,
        out_specs=pl.BlockSpec((blk, H), lambda i: (i, 0)),
        out_shape=jax.ShapeDtypeStruct((n, H), _F32),
    )(*args)


def _selu(x):
    return 1.0507009873554805 * jnp.where(
        x > 0, x, 1.6732632423543772 * (jnp.exp(x) - 1.0)
    )


def _pred_mlp(xs, xd, w1, b1, w2, b2, w3, b3, blk):
    n = xs.shape[0]

    def body(xs_r, xd_r, w1_r, b1_r, w2_r, b2_r, w3_r, b3_r, o_r):
        w1v = w1_r[...]
        x = _dot(xs_r[...], w1v[0:H]) + _dot(xd_r[...], w1v[H : 2 * H]) + b1_r[...]
        x = _selu(x)
        x = _selu(_dot(x, w2_r[...]) + b2_r[...])
        o_r[...] = _dot(x, w3_r[...]) + b3_r[...]

    return pl.pallas_call(
        body,
        grid=(n // blk,),
        in_specs=[
            pl.BlockSpec((blk, H), lambda i: (i, 0)),
            pl.BlockSpec((blk, H), lambda i: (i, 0)),
            pl.BlockSpec((2 * H, 16), lambda i: (0, 0)),
            pl.BlockSpec((1, 16), lambda i: (0, 0)),
            pl.BlockSpec((16, 8), lambda i: (0, 0)),
            pl.BlockSpec((1, 8), lambda i: (0, 0)),
            pl.BlockSpec((8, 1), lambda i: (0, 0)),
            pl.BlockSpec((1, 1), lambda i: (0, 0)),
        ],
        out_specs=pl.BlockSpec((blk, 1), lambda i: (i, 0)),
        out_shape=jax.ShapeDtypeStruct((n, 1), _F32),
    )(
        xs,
        xd,
        w1,
        b1.reshape(1, 16),
        w2,
        b2.reshape(1, 8),
        w3,
        b3.reshape(1, 1),
    )


def _pad_idx(a, m, fill):
    e = a.shape[0]
    ep = _ceil_to(e, m)
    if ep != e:
        a = jnp.concatenate([a, jnp.full((ep - e,), fill, _I32)])
    return a


def _pack_sd(src, dst, m):
    s = _pad_idx(src, m, 0).reshape(-1, 1, CH)
    d = _pad_idx(dst, m, _FILL).reshape(-1, 1, CH)
    return jnp.concatenate([s, d], axis=1).reshape(-1)


def kernel(
    customer_feats,
    product_feats,
    group_feats,
    edge_cp_src,
    edge_cp_dst,
    edge_pg_src,
    edge_pg_dst,
    pos_src,
    pos_dst,
    neg_src,
    neg_dst,
    params,
):
    p = params
    n_c = customer_feats.shape[0]
    n_p = product_feats.shape[0]
    n_g = group_feats.shape[0]

    cp_s = edge_cp_src.astype(_I32)
    cp_d = edge_cp_dst.astype(_I32)
    pg_s = edge_pg_src.astype(_I32)
    pg_d = edge_pg_dst.astype(_I32)

    m = NS * CH * 2
    sd_cp = _pack_sd(cp_s, cp_d, m)
    sd_pc = _pack_sd(cp_d, cp_s, m)
    sd_gp = _pack_sd(pg_d, pg_s, m)
    sd_pg = _pack_sd(pg_s, pg_d, m)
    cp_dF = _pad_idx(cp_d, m, _FILL)
    cp_sF = _pad_idx(cp_s, m, _FILL)
    pg_sF = _pad_idx(pg_s, m, _FILL)
    pg_dF = _pad_idx(pg_d, m, _FILL)

    zrow_h = jnp.zeros((CH, H), _F32)
    zrow_l = jnp.zeros((CH, L), _F32)

    h_c = _embed(customer_feats, p["Wc1"], p["bc1"], p["Wc2"], p["bc2"], 2000)
    h_p = _embed(product_feats, p["Wp1e"], p["bp1e"], p["Wp2e"], p["bp2e"], 2000)
    h_g = _embed(group_feats, p["Wg1"], p["bg1"], p["Wg2"], p["bg2"], 1000)

    dir_shapes = (
        (n_p, cp_dF.shape[0]),
        (n_c, cp_sF.shape[0]),
        (n_p, pg_sF.shape[0]),
        (n_g, pg_dF.shape[0]),
    )
    cnt_kernel, rowoffs = _make_counts(dir_shapes)
    cnt_packed = cnt_kernel(cp_dF, cp_sF, pg_sF, pg_dF, zrow_l)
    cnt_flat = cnt_packed.reshape(-1)
    counts = []
    for (n_dst, _), ro in zip(dir_shapes, rowoffs):
        mrows = -(-n_dst // L)
        counts.append(cnt_flat[ro * L : ro * L + mrows * L][:n_dst].reshape(n_dst, 1))
    cnt_cp, cnt_pc, cnt_gp, cnt_pg = counts

    for l in range(2):
        s_cp = _segsum(h_c, sd_cp, n_p, zrow_h)
        s_gp = _segsum(h_g, sd_gp, n_p, zrow_h)
        s_pc = _segsum(h_p, sd_pc, n_c, zrow_h)
        s_pg = _segsum(h_p, sd_pg, n_g, zrow_h)
        z_p = _combine(
            h_p,
            [
                (s_cp, cnt_cp, p["Ws%d_cp" % l], p["Wn%d_cp" % l]),
                (s_gp, cnt_gp, p["Ws%d_gp" % l], p["Wn%d_gp" % l]),
            ],
            2000,
        )
        z_c = _combine(
            h_c, [(s_pc, cnt_pc, p["Ws%d_pc" % l], p["Wn%d_pc" % l])], 2000
        )
        z_g = _combine(
            h_g, [(s_pg, cnt_pg, p["Ws%d_pg" % l], p["Wn%d_pg" % l])], 1000
        )
        h_c, h_p, h_g = z_c, z_p, z_g

    e_pos = pos_src.shape[0]
    e_neg = neg_src.shape[0]
    mg = NS * NC * CH * 2
    ps = _pad_idx(pos_src.astype(_I32), mg, 0)
    pd = _pad_idx(pos_dst.astype(_I32), mg, 0)
    ns_ = _pad_idx(neg_src.astype(_I32), mg, 0)
    nd = _pad_idx(neg_dst.astype(_I32), mg, 0)
    e_pad = ps.shape[0]
    g1, g2, g3, g4 = _make_gather4(n_c, n_p, e_pad)(h_c, h_p, ps, pd, ns_, nd)

    pos = _pred_mlp(
        g1, g2, p["Wq1"], p["bq1"], p["Wq2"], p["bq2"], p["Wq3"], p["bq3"], 2048
    )[:e_pos]
    neg = _pred_mlp(
        g3, g4, p["Wq1"], p["bq1"], p["Wq2"], p["bq2"], p["Wq3"], p["bq3"], 2048
    )[:e_neg]

    return h_c, h_p, h_g, pos, neg
